# Initial kernel scaffold; baseline (speedup 1.0000x reference)
import jax
import jax.numpy as jnp
from jax.experimental import pallas as pl


def kernel(x, table, W, b):
    raise NotImplementedError("write your pallas kernel here")


# speedup vs baseline: 18.2490x; 18.2490x over previous
import functools

import jax
import jax.numpy as jnp
from jax import lax
from jax.experimental import pallas as pl
from jax.experimental.pallas import tpu as pltpu
from jax.experimental.pallas import tpu_sc as plsc

_V = 100000
_E = 64
_B = 4096
_H = 200

_NC = 2
_NS = 16
_NW = _NC * _NS
_LANES = 16

_ROWS_PER_W = _B // _NW
_CH = 64
_N_CHUNKS = _ROWS_PER_W // _CH
_VPP = 2 * _H // _LANES


def _stage1_body(tbl_ref, w_ref, out_ref):
    out_ref[...] = jnp.sum(tbl_ref[...] * w_ref[...], axis=1, keepdims=True)


def _stage1(table, w_scaled):
    blk = 2000
    return pl.pallas_call(
        _stage1_body,
        grid=(_V // blk,),
        in_specs=[
            pl.BlockSpec((blk, _E), lambda i: (i, 0)),
            pl.BlockSpec((1, _E), lambda i: (0, 0)),
        ],
        out_specs=pl.BlockSpec((blk, 1), lambda i: (i, 0)),
        out_shape=jax.ShapeDtypeStruct((_V, 1), jnp.float32),
    )(table, w_scaled)


def _sc_body(t_hbm, xf_hbm, b_hbm, out_hbm, t_v, idx_v, res_v, b_v):
    wid = lax.axis_index("s") * _NC + lax.axis_index("c")

    pltpu.sync_copy(t_hbm, t_v)
    pltpu.sync_copy(b_hbm, b_v)

    lane = lax.iota(jnp.int32, _LANES)
    mask_a = jnp.where(lane < (_H - (_VPP // 2) * _LANES), 1.0, 0.0)
    mask_b = 1.0 - mask_a

    for c in range(_N_CHUNKS):
        base_row = wid * _ROWS_PER_W + c * _CH
        pltpu.sync_copy(xf_hbm.at[pl.ds(base_row * _H, _CH * _H)], idx_v)

        def pair_body(p, _, c=c):
            base = p * (2 * _H)
            acc_a = jnp.zeros((_LANES,), jnp.float32)
            acc_b = jnp.zeros((_LANES,), jnp.float32)
            for j in range(_VPP // 2):
                idx = idx_v[pl.ds(base + j * _LANES, _LANES)]
                acc_a = acc_a + plsc.load_gather(t_v, [idx])
            idx = idx_v[pl.ds(base + (_VPP // 2) * _LANES, _LANES)]
            g = plsc.load_gather(t_v, [idx])
            acc_a = acc_a + g * mask_a
            acc_b = acc_b + g * mask_b
            for j in range(_VPP // 2 + 1, _VPP):
                idx = idx_v[pl.ds(base + j * _LANES, _LANES)]
                acc_b = acc_b + plsc.load_gather(t_v, [idx])
            r = c * _CH + 2 * p
            rvec = jnp.full((_LANES,), r, jnp.int32)
            last = lane == (_LANES - 1)
            plsc.store_scatter(res_v, [rvec], plsc.cumsum(acc_a), mask=last)
            plsc.store_scatter(res_v, [rvec + 1], plsc.cumsum(acc_b), mask=last)
            return _

        lax.fori_loop(0, _CH // 2, pair_body, None)

    for k in range(_ROWS_PER_W // _LANES):
        z = res_v[pl.ds(k * _LANES, _LANES)] + b_v[...]
        res_v[pl.ds(k * _LANES, _LANES)] = 1.0 / (1.0 + jnp.exp(-z))

    pltpu.sync_copy(res_v, out_hbm.at[pl.ds(wid * _ROWS_PER_W, _ROWS_PER_W)])


@functools.cache
def _sc_stage2():
    return pl.kernel(
        _sc_body,
        out_type=jax.ShapeDtypeStruct((_B,), jnp.float32),
        mesh=plsc.VectorSubcoreMesh(
            core_axis_name="c", subcore_axis_name="s", num_cores=_NC, num_subcores=_NS
        ),
        scratch_types=[
            pltpu.VMEM((_V,), jnp.float32),
            pltpu.VMEM((_CH * _H,), jnp.int32),
            pltpu.VMEM((_ROWS_PER_W,), jnp.float32),
            pltpu.VMEM((_LANES,), jnp.float32),
        ],
        compiler_params=pltpu.CompilerParams(
            use_tc_tiling_on_sc=False, needs_layout_passes=False
        ),
    )


@jax.jit
def kernel(x, table, W, b):
    w_scaled = (W.astype(jnp.float32) * (1.0 / _H)).reshape(1, _E)
    t = _stage1(table, w_scaled).reshape(-1)
    b16 = jnp.broadcast_to(b.reshape(1), (_LANES,)).astype(jnp.float32)
    xf = x.astype(jnp.int32).reshape(-1)
    y = _sc_stage2()(t, xf, b16)
    return y.reshape(_B, 1)


# speedup vs baseline: 21.8870x; 1.1994x over previous
import functools

import jax
import jax.numpy as jnp
from jax import lax
from jax.experimental import pallas as pl
from jax.experimental.pallas import tpu as pltpu
from jax.experimental.pallas import tpu_sc as plsc

_V = 100000
_E = 64
_B = 4096
_H = 200

_NC = 2
_NS = 16
_NW = _NC * _NS
_LANES = 16

_ROWS_PER_W = _B // _NW
_GROUPS = _ROWS_PER_W // _LANES


def _stage1_body(tbl_ref, w_ref, out_ref):
    out_ref[...] = jnp.sum(tbl_ref[...] * w_ref[...], axis=1)


def _stage1(table, w_scaled):
    blk = 8192
    return pl.pallas_call(
        _stage1_body,
        grid=(pl.cdiv(_V, blk),),
        in_specs=[
            pl.BlockSpec((blk, _E), lambda i: (i, 0)),
            pl.BlockSpec((1, _E), lambda i: (0, 0)),
        ],
        out_specs=pl.BlockSpec((blk,), lambda i: (i,)),
        out_shape=jax.ShapeDtypeStruct((_V,), jnp.float32),
    )(table, w_scaled)


def _sc_body(t_hbm, x_hbm, b_hbm, out_hbm, t_v, idx_v, res_v, b_v, sem_t, sem_x):
    wid = lax.axis_index("s") * _NC + lax.axis_index("c")
    base_row = wid * _ROWS_PER_W

    ct = pltpu.async_copy(t_hbm, t_v, sem_t)
    cx = pltpu.async_copy(x_hbm.at[pl.ds(base_row, _ROWS_PER_W)], idx_v, sem_x)
    pltpu.sync_copy(b_hbm, b_v)
    cx.wait()
    ct.wait()

    bias = b_v[...]
    for g in range(_GROUPS):
        rows = jnp.full((_LANES,), g * _LANES, jnp.int32) + lax.iota(jnp.int32, _LANES)

        def tok_body(l, acc, rows=rows):
            col = jnp.full((_LANES,), l, jnp.int32)
            iv = plsc.load_gather(idx_v, [rows, col])
            return acc + plsc.load_gather(t_v, [iv])

        acc = lax.fori_loop(
            0, _H, tok_body, jnp.zeros((_LANES,), jnp.float32), unroll=8
        )
        z = acc + bias
        res_v[pl.ds(g * _LANES, _LANES)] = 1.0 / (1.0 + jnp.exp(-z))

    pltpu.sync_copy(res_v, out_hbm.at[pl.ds(base_row, _ROWS_PER_W)])


@functools.cache
def _sc_stage2():
    return pl.kernel(
        _sc_body,
        out_type=jax.ShapeDtypeStruct((_B,), jnp.float32),
        mesh=plsc.VectorSubcoreMesh(
            core_axis_name="c", subcore_axis_name="s", num_cores=_NC, num_subcores=_NS
        ),
        scratch_types=[
            pltpu.VMEM((_V,), jnp.float32),
            pltpu.VMEM((_ROWS_PER_W, _H), jnp.int32),
            pltpu.VMEM((_ROWS_PER_W,), jnp.float32),
            pltpu.VMEM((_LANES,), jnp.float32),
            pltpu.SemaphoreType.DMA,
            pltpu.SemaphoreType.DMA,
        ],
        compiler_params=pltpu.CompilerParams(
            use_tc_tiling_on_sc=False, needs_layout_passes=False
        ),
    )


@jax.jit
def kernel(x, table, W, b):
    w_scaled = (W.astype(jnp.float32) * (1.0 / _H)).reshape(1, _E)
    t = _stage1(table, w_scaled)
    b16 = jnp.broadcast_to(b.reshape(1), (_LANES,)).astype(jnp.float32)
    y = _sc_stage2()(t, x.astype(jnp.int32), b16)
    return y.reshape(_B, 1)


# speedup vs baseline: 53.3802x; 2.4389x over previous
import functools

import jax
import jax.numpy as jnp
from jax import lax
from jax.experimental import pallas as pl
from jax.experimental.pallas import tpu as pltpu
from jax.experimental.pallas import tpu_sc as plsc

_V = 100000
_E = 64
_B = 4096
_H = 200

_NC = 2
_NS = 16
_NW = _NC * _NS
_LANES = 16

_ROWS_PER_W = _B // _NW
_GROUPS = _ROWS_PER_W // _LANES


def _stage1_body(tblT_ref, w_ref, out_ref):
    out_ref[...] = jnp.sum(tblT_ref[...] * w_ref[...], axis=0)


def _stage1(tableT, w_scaled):
    blk = 8192
    return pl.pallas_call(
        _stage1_body,
        grid=(pl.cdiv(_V, blk),),
        in_specs=[
            pl.BlockSpec((_E, blk), lambda i: (0, i)),
            pl.BlockSpec((_E, 1), lambda i: (0, 0)),
        ],
        out_specs=pl.BlockSpec((blk,), lambda i: (i,)),
        out_shape=jax.ShapeDtypeStruct((_V,), jnp.float32),
    )(tableT, w_scaled)


def _sc_body(t_hbm, xT_hbm, b_hbm, out_hbm, t_v, idx_v, res_v, b_v, sem_t, sem_x):
    wid = lax.axis_index("s") * _NC + lax.axis_index("c")
    base_row = wid * _ROWS_PER_W

    ct = pltpu.async_copy(t_hbm, t_v, sem_t)
    cx = pltpu.async_copy(xT_hbm.at[:, pl.ds(base_row, _ROWS_PER_W)], idx_v, sem_x)
    pltpu.sync_copy(b_hbm, b_v)
    cx.wait()
    ct.wait()

    bias = b_v[...]
    for g in range(_GROUPS):

        def tok_body(l, acc, g=g):
            iv = idx_v[l, pl.ds(g * _LANES, _LANES)]
            return acc + plsc.load_gather(t_v, [iv])

        acc = lax.fori_loop(
            0, _H, tok_body, jnp.zeros((_LANES,), jnp.float32), unroll=8
        )
        z = acc + bias
        res_v[pl.ds(g * _LANES, _LANES)] = 1.0 / (1.0 + jnp.exp(-z))

    pltpu.sync_copy(res_v, out_hbm.at[pl.ds(base_row, _ROWS_PER_W)])


@functools.cache
def _sc_stage2():
    return pl.kernel(
        _sc_body,
        out_type=jax.ShapeDtypeStruct((_B,), jnp.float32),
        mesh=plsc.VectorSubcoreMesh(
            core_axis_name="c", subcore_axis_name="s", num_cores=_NC, num_subcores=_NS
        ),
        scratch_types=[
            pltpu.VMEM((_V,), jnp.float32),
            pltpu.VMEM((_H, _ROWS_PER_W), jnp.int32),
            pltpu.VMEM((_ROWS_PER_W,), jnp.float32),
            pltpu.VMEM((_LANES,), jnp.float32),
            pltpu.SemaphoreType.DMA,
            pltpu.SemaphoreType.DMA,
        ],
        compiler_params=pltpu.CompilerParams(
            use_tc_tiling_on_sc=False, needs_layout_passes=False
        ),
    )


@jax.jit
def kernel(x, table, W, b):
    w_scaled = (W.astype(jnp.float32) * (1.0 / _H)).reshape(_E, 1)
    t = _stage1(table.T, w_scaled)
    b16 = jnp.broadcast_to(b.reshape(1), (_LANES,)).astype(jnp.float32)
    y = _sc_stage2()(t, x.T.astype(jnp.int32), b16)
    return y.reshape(_B, 1)


# speedup vs baseline: 58.2691x; 1.0916x over previous
import functools

import jax
import jax.numpy as jnp
from jax import lax
from jax.experimental import pallas as pl
from jax.experimental.pallas import tpu as pltpu
from jax.experimental.pallas import tpu_sc as plsc

_V = 100000
_E = 64
_B = 4096
_H = 200

_NC = 2
_NS = 16
_NW = _NC * _NS
_LANES = 16

_ROWS_PER_W = _B // _NW
_GROUPS = _ROWS_PER_W // _LANES


def _stage1_body(b_ref, tblT_ref, w_ref, out_ref):
    out_ref[...] = jnp.sum(tblT_ref[...] * w_ref[...], axis=0) + b_ref[0]


def _stage1(tableT, w_scaled, b_scaled):
    blk = 25600
    return pl.pallas_call(
        _stage1_body,
        grid=(pl.cdiv(_V, blk),),
        in_specs=[
            pl.BlockSpec(memory_space=pltpu.SMEM),
            pl.BlockSpec((_E, blk), lambda i: (0, i)),
            pl.BlockSpec((_E, 1), lambda i: (0, 0)),
        ],
        out_specs=pl.BlockSpec((blk,), lambda i: (i,)),
        out_shape=jax.ShapeDtypeStruct((_V,), jnp.float32),
    )(b_scaled, tableT, w_scaled)


def _sc_body(t_hbm, xT_hbm, out_hbm, t_v, idx_v, res_v, sem_t, sem_x):
    wid = lax.axis_index("s") * _NC + lax.axis_index("c")
    base_row = wid * _ROWS_PER_W

    ct = pltpu.async_copy(t_hbm, t_v, sem_t)
    cx = pltpu.async_copy(xT_hbm.at[:, pl.ds(base_row, _ROWS_PER_W)], idx_v, sem_x)
    cx.wait()
    ct.wait()

    for g in range(_GROUPS):

        def tok_body(l, acc, g=g):
            iv = idx_v[l, pl.ds(g * _LANES, _LANES)]
            return acc + plsc.load_gather(t_v, [iv])

        z = lax.fori_loop(
            0, _H, tok_body, jnp.zeros((_LANES,), jnp.float32), unroll=8
        )
        res_v[pl.ds(g * _LANES, _LANES)] = 1.0 / (1.0 + jnp.exp(-z))

    pltpu.sync_copy(res_v, out_hbm.at[pl.ds(base_row, _ROWS_PER_W)])


@functools.cache
def _sc_stage2():
    return pl.kernel(
        _sc_body,
        out_type=jax.ShapeDtypeStruct((_B,), jnp.float32),
        mesh=plsc.VectorSubcoreMesh(
            core_axis_name="c", subcore_axis_name="s", num_cores=_NC, num_subcores=_NS
        ),
        scratch_types=[
            pltpu.VMEM((_V,), jnp.float32),
            pltpu.VMEM((_H, _ROWS_PER_W), jnp.int32),
            pltpu.VMEM((_ROWS_PER_W,), jnp.float32),
            pltpu.SemaphoreType.DMA,
            pltpu.SemaphoreType.DMA,
        ],
        compiler_params=pltpu.CompilerParams(
            use_tc_tiling_on_sc=False, needs_layout_passes=False
        ),
    )


@jax.jit
def kernel(x, table, W, b):
    w_scaled = (W.astype(jnp.float32) * (1.0 / _H)).reshape(_E, 1)
    b_scaled = b.astype(jnp.float32) * (1.0 / _H)
    t = _stage1(table.T, w_scaled, b_scaled)
    y = _sc_stage2()(t, x.T.astype(jnp.int32))
    return y.reshape(_B, 1)


# speedup vs baseline: 63.6782x; 1.0928x over previous
import functools

import jax
import jax.numpy as jnp
from jax import lax
from jax.experimental import pallas as pl
from jax.experimental.pallas import tpu as pltpu
from jax.experimental.pallas import tpu_sc as plsc

_V = 100000
_E = 64
_B = 4096
_H = 200

_NC = 2
_NS = 16
_NW = _NC * _NS
_LANES = 16

_ROWS_PER_W = _B // _NW
_GROUPS = _ROWS_PER_W // _LANES


def _stage1_body(b_ref, tblT_ref, w_ref, out_ref):
    out_ref[...] = jnp.sum(tblT_ref[...] * w_ref[...], axis=0) + b_ref[0]


def _stage1(tableT, w_scaled, b_scaled):
    blk = 25600
    return pl.pallas_call(
        _stage1_body,
        grid=(pl.cdiv(_V, blk),),
        in_specs=[
            pl.BlockSpec(memory_space=pltpu.SMEM),
            pl.BlockSpec((_E, blk), lambda i: (0, i)),
            pl.BlockSpec((_E, 1), lambda i: (0, 0)),
        ],
        out_specs=pl.BlockSpec((blk,), lambda i: (i,)),
        out_shape=jax.ShapeDtypeStruct((_V,), jnp.float32),
    )(b_scaled, tableT, w_scaled)


def _sc_body(t_hbm, x4_hbm, out_hbm, t_v, idx_v, res_v, sem_t, sem_x):
    wid = lax.axis_index("s") * _NC + lax.axis_index("c")
    base_row = wid * _ROWS_PER_W

    ct = pltpu.async_copy(t_hbm, t_v, sem_t)
    cx = pltpu.async_copy(x4_hbm.at[:, wid], idx_v, sem_x)
    cx.wait()
    ct.wait()

    for g in range(_GROUPS):

        def tok_body(lh, acc, g=g):
            for ll in range(8):
                iv = idx_v[lh, ll, pl.ds(g * _LANES, _LANES)]
                acc = acc + plsc.load_gather(t_v, [iv])
            return acc

        z = lax.fori_loop(
            0, _H // 8, tok_body, jnp.zeros((_LANES,), jnp.float32)
        )
        res_v[pl.ds(g * _LANES, _LANES)] = 1.0 / (1.0 + jnp.exp(-z))

    pltpu.sync_copy(res_v, out_hbm.at[pl.ds(base_row, _ROWS_PER_W)])


@functools.cache
def _sc_stage2():
    return pl.kernel(
        _sc_body,
        out_type=jax.ShapeDtypeStruct((_B,), jnp.float32),
        mesh=plsc.VectorSubcoreMesh(
            core_axis_name="c", subcore_axis_name="s", num_cores=_NC, num_subcores=_NS
        ),
        scratch_types=[
            pltpu.VMEM((_V,), jnp.float32),
            pltpu.VMEM((_H // 8, 8, _ROWS_PER_W), jnp.int32),
            pltpu.VMEM((_ROWS_PER_W,), jnp.float32),
            pltpu.SemaphoreType.DMA,
            pltpu.SemaphoreType.DMA,
        ],
        compiler_params=pltpu.CompilerParams(
            use_tc_tiling_on_sc=False, needs_layout_passes=False
        ),
    )


@jax.jit
def kernel(x, table, W, b):
    w_scaled = (W.astype(jnp.float32) * (1.0 / _H)).reshape(_E, 1)
    b_scaled = b.astype(jnp.float32) * (1.0 / _H)
    t = _stage1(table.T, w_scaled, b_scaled)
    x4 = (
        x.astype(jnp.int32)
        .T.reshape(_H // 8, 8, _NW, _ROWS_PER_W)
        .transpose(0, 2, 1, 3)
    )
    y = _sc_stage2()(t, x4)
    return y.reshape(_B, 1)


# speedup vs baseline: 63.6827x; 1.0001x over previous
import functools

import jax
import jax.numpy as jnp
from jax import lax
from jax.experimental import pallas as pl
from jax.experimental.pallas import tpu as pltpu
from jax.experimental.pallas import tpu_sc as plsc

_V = 100000
_E = 64
_B = 4096
_H = 200

_NC = 2
_NS = 16
_NW = _NC * _NS
_LANES = 16

_ROWS_PER_W = _B // _NW
_GROUPS = _ROWS_PER_W // _LANES


def _stage1_body(b_ref, tblT_ref, w_ref, out_ref):
    out_ref[...] = jnp.sum(tblT_ref[...] * w_ref[...], axis=0) + b_ref[0]


def _stage1(tableT, w_scaled, b_scaled):
    blk = 25600
    return pl.pallas_call(
        _stage1_body,
        grid=(pl.cdiv(_V, blk),),
        in_specs=[
            pl.BlockSpec(memory_space=pltpu.SMEM),
            pl.BlockSpec((_E, blk), lambda i: (0, i)),
            pl.BlockSpec((_E, 1), lambda i: (0, 0)),
        ],
        out_specs=pl.BlockSpec((blk,), lambda i: (i,)),
        out_shape=jax.ShapeDtypeStruct((_V,), jnp.float32),
    )(b_scaled, tableT, w_scaled)


def _sc_body(t_hbm, x4_hbm, out_hbm, t_v, idx_v, res_v, sem_t, sem_x):
    wid = lax.axis_index("s") * _NC + lax.axis_index("c")
    base_row = wid * _ROWS_PER_W

    nchunks = 4
    chunk = _V // nchunks
    cts = [
        pltpu.async_copy(
            t_hbm.at[pl.ds(i * chunk, chunk)], t_v.at[pl.ds(i * chunk, chunk)], sem_t
        )
        for i in range(nchunks)
    ]
    cx = pltpu.async_copy(x4_hbm.at[:, wid], idx_v, sem_x)
    cx.wait()
    for c in cts:
        c.wait()

    for g in range(_GROUPS):

        def tok_body(lh, acc, g=g):
            for ll in range(8):
                iv = idx_v[lh, ll, pl.ds(g * _LANES, _LANES)]
                acc = acc + plsc.load_gather(t_v, [iv])
            return acc

        z = lax.fori_loop(
            0, _H // 8, tok_body, jnp.zeros((_LANES,), jnp.float32)
        )
        res_v[pl.ds(g * _LANES, _LANES)] = 1.0 / (1.0 + jnp.exp(-z))

    pltpu.sync_copy(res_v, out_hbm.at[pl.ds(base_row, _ROWS_PER_W)])


@functools.cache
def _sc_stage2():
    return pl.kernel(
        _sc_body,
        out_type=jax.ShapeDtypeStruct((_B,), jnp.float32),
        mesh=plsc.VectorSubcoreMesh(
            core_axis_name="c", subcore_axis_name="s", num_cores=_NC, num_subcores=_NS
        ),
        scratch_types=[
            pltpu.VMEM((_V,), jnp.float32),
            pltpu.VMEM((_H // 8, 8, _ROWS_PER_W), jnp.int32),
            pltpu.VMEM((_ROWS_PER_W,), jnp.float32),
            pltpu.SemaphoreType.DMA,
            pltpu.SemaphoreType.DMA,
        ],
        compiler_params=pltpu.CompilerParams(
            use_tc_tiling_on_sc=False, needs_layout_passes=False
        ),
    )


@jax.jit
def kernel(x, table, W, b):
    w_scaled = (W.astype(jnp.float32) * (1.0 / _H)).reshape(_E, 1)
    b_scaled = b.astype(jnp.float32) * (1.0 / _H)
    t = _stage1(table.T, w_scaled, b_scaled)
    x4 = (
        x.astype(jnp.int32)
        .T.reshape(_H // 8, 8, _NW, _ROWS_PER_W)
        .transpose(0, 2, 1, 3)
    )
    y = _sc_stage2()(t, x4)
    return y.reshape(_B, 1)
